# baseline (speedup 1.0000x reference)
import functools

import jax
import jax.numpy as jnp
from jax import lax
from jax.experimental import pallas as pl
from jax.experimental.pallas import tpu as pltpu
from jax.experimental.pallas import tpu_sc as plsc

_VOCAB = 1000000
_D = 64
_PAD = 0

_B_ROWS = 4096
_SEQ = 200
_B = _B_ROWS * _SEQ

_NC = 2
_NS = 16
_NW = _NC * _NS
_CHUNK = 128
_PER_W = _B // _NW
_N_CHUNKS = _PER_W // _CHUNK


def _gather_kernel(idx_hbm, table_hbm, out_hbm, idx_v, rows_v, sem):
    wid = lax.axis_index("s") * _NC + lax.axis_index("c")
    base = wid * _PER_W
    pltpu.sync_copy(idx_hbm.at[wid], idx_v)

    def body(g, carry):
        pltpu.async_copy(table_hbm.at[idx_v.at[g]], rows_v, sem).wait()
        pltpu.sync_copy(rows_v, out_hbm.at[pl.ds(base + g * _CHUNK, _CHUNK)])
        return carry

    lax.fori_loop(0, _N_CHUNKS, body, 0)


@jax.jit
def _sc_gather(idx3, table):
    mesh = plsc.VectorSubcoreMesh(core_axis_name="c", subcore_axis_name="s")
    f = functools.partial(
        pl.kernel,
        out_type=jax.ShapeDtypeStruct((_B, _D), jnp.float32),
        mesh=mesh,
        scratch_types=[
            pltpu.VMEM((_N_CHUNKS, _CHUNK), jnp.int32),
            pltpu.VMEM((_CHUNK, _D), jnp.float32),
            pltpu.SemaphoreType.DMA,
        ],
        compiler_params=pltpu.CompilerParams(use_tc_tiling_on_sc=False),
    )(_gather_kernel)
    return f(idx3, table)


def _mask_body(x_ref, o_ref):
    o_ref[...] = x_ref[...] == _PAD


@jax.jit
def _tc_mask(x):
    return pl.pallas_call(
        _mask_body,
        out_shape=jax.ShapeDtypeStruct((_B_ROWS, _SEQ), jnp.bool_),
    )(x)


def kernel(x, table):
    idx3 = x.reshape(_NW, _N_CHUNKS, _CHUNK)
    h = _sc_gather(idx3, table).reshape(_B_ROWS, _SEQ, _D)
    mask = _tc_mask(x)
    return (h, mask)


# speedup vs baseline: 1.1169x; 1.1169x over previous
import functools

import jax
import jax.numpy as jnp
from jax import lax
from jax.experimental import pallas as pl
from jax.experimental.pallas import tpu as pltpu
from jax.experimental.pallas import tpu_sc as plsc

_VOCAB = 1000000
_D = 64
_PAD = 0

_B_ROWS = 4096
_SEQ = 200
_B = _B_ROWS * _SEQ

_NC = 2
_NS = 16
_NW = _NC * _NS
_IDX_W = 128
_GPC = 4
_CHUNK = _IDX_W * _GPC
_PER_W = _B // _NW
_N_IDX_ROWS = _PER_W // _IDX_W
_N_CHUNKS = _PER_W // _CHUNK
_NBUF = 2


def _gather_kernel(idx_hbm, table_hbm, out_hbm, idx_v, bufs, gsems, osems):
    wid = lax.axis_index("s") * _NC + lax.axis_index("c")
    base = wid * _PER_W
    pltpu.sync_copy(idx_hbm.at[wid], idx_v)

    def fire_gathers(t, b):
        for j in range(_GPC):
            pltpu.async_copy(
                table_hbm.at[idx_v.at[t * _GPC + j]],
                bufs[b].at[pl.ds(j * _IDX_W, _IDX_W)],
                gsems[b],
            )

    def wait_gathers(t, b):
        for j in range(_GPC):
            pltpu.make_async_copy(
                table_hbm.at[idx_v.at[t * _GPC + j]],
                bufs[b].at[pl.ds(j * _IDX_W, _IDX_W)],
                gsems[b],
            ).wait()

    def out_copy(t, b):
        return (
            bufs[b],
            out_hbm.at[pl.ds(base + t * _CHUNK, _CHUNK)],
            osems[b],
        )

    fire_gathers(0, 0)

    @pl.loop(0, _N_CHUNKS // _NBUF)
    def _(t2):
        for b in range(_NBUF):
            t = _NBUF * t2 + b
            wait_gathers(t, b)
            nb = (b + 1) % _NBUF

            @pl.when(jnp.logical_and(t >= 1, t + 1 < _N_CHUNKS))
            def _():
                pltpu.make_async_copy(*out_copy(t - 1, nb)).wait()

            @pl.when(t + 1 < _N_CHUNKS)
            def _():
                fire_gathers(t + 1, nb)

            pltpu.async_copy(*out_copy(t, b))

    pltpu.make_async_copy(*out_copy(_N_CHUNKS - 2, (_N_CHUNKS - 2) % _NBUF)).wait()
    pltpu.make_async_copy(*out_copy(_N_CHUNKS - 1, (_N_CHUNKS - 1) % _NBUF)).wait()


@jax.jit
def _sc_gather(idx3, table):
    mesh = plsc.VectorSubcoreMesh(core_axis_name="c", subcore_axis_name="s")
    f = functools.partial(
        pl.kernel,
        out_type=jax.ShapeDtypeStruct((_B, _D), jnp.float32),
        mesh=mesh,
        scratch_types=[
            pltpu.VMEM((_N_IDX_ROWS, _IDX_W), jnp.int32),
            [pltpu.VMEM((_CHUNK, _D), jnp.float32) for _ in range(_NBUF)],
            [pltpu.SemaphoreType.DMA for _ in range(_NBUF)],
            [pltpu.SemaphoreType.DMA for _ in range(_NBUF)],
        ],
        compiler_params=pltpu.CompilerParams(use_tc_tiling_on_sc=False),
    )(_gather_kernel)
    return f(idx3, table)


def _mask_body(x_ref, o_ref):
    o_ref[...] = x_ref[...] == _PAD


@jax.jit
def _tc_mask(x):
    return pl.pallas_call(
        _mask_body,
        out_shape=jax.ShapeDtypeStruct((_B_ROWS, _SEQ), jnp.bool_),
    )(x)


def kernel(x, table):
    idx3 = x.reshape(_NW, _N_IDX_ROWS, _IDX_W)
    h = _sc_gather(idx3, table).reshape(_B_ROWS, _SEQ, _D)
    mask = _tc_mask(x)
    return (h, mask)
